# baseline (speedup 1.0000x reference)
import functools

import jax
import jax.numpy as jnp
from jax import lax
from jax.experimental import pallas as pl
from jax.experimental.pallas import tpu as pltpu
from jax.experimental.pallas import tpu_sc as plsc

B, H, W, C = 4, 224, 224, 96
P = B * H * W
D = 4 * C
CHUNK = 128
NW = 32
CHUNKS = P // CHUNK
CPB = (H * W) // CHUNK
CPW = CHUNKS // NW
NG = CHUNK // 16
NBUF = 2
SPLIT = 64

_mesh = plsc.VectorSubcoreMesh(core_axis_name="c", subcore_axis_name="s")


def _stn_body(im_ref, xs_ref, ys_ref, out_ref,
              idx_vs, gbufs, obufs, x_vs, y_vs, gsems, osems):
    wid = lax.axis_index("s") * 2 + lax.axis_index("c")

    def cell(p, base, g):
        sl = pl.ds(g * 16, 16)
        x = jnp.clip(x_vs[p][sl], -1.0, 1.0)
        y = jnp.clip(y_vs[p][sl], -1.0, 1.0)
        px = (x + 1.0) * ((W - 1) * 0.5)
        py = (y + 1.0) * ((H - 1) * 0.5)
        x0 = px.astype(jnp.int32)
        y0 = py.astype(jnp.int32)
        fx = px - x0.astype(jnp.float32)
        fy = py - y0.astype(jnp.float32)
        return base + y0 * W + x0, fx, fy

    def fire(p, ci):
        cid = wid * CPW + ci
        base = lax.div(cid, CPB) * (H * W)
        sl_in = pl.ds(cid * CHUNK, CHUNK)
        pltpu.sync_copy(xs_ref.at[sl_in], x_vs[p])
        pltpu.sync_copy(ys_ref.at[sl_in], y_vs[p])
        for g in range(NG):
            idx, _, _ = cell(p, base, g)
            k, off = divmod(g * 16, SPLIT)
            idx_vs[p][k][pl.ds(off, 16)] = idx
        for k in range(2):
            pltpu.async_copy(im_ref.at[idx_vs[p][k]], gbufs[p][k], gsems[p])

    def drain_blend(p, ci, first):
        for k in range(2):
            pltpu.make_async_copy(im_ref.at[idx_vs[p][k]], gbufs[p][k],
                                  gsems[p]).wait()
        cid = wid * CPW + ci
        out_slice = out_ref.at[pl.ds(cid * CHUNK, CHUNK)]

        @pl.when(jnp.logical_not(first))
        def _():
            pltpu.make_async_copy(obufs[p], out_slice, osems[p]).wait()
        base = lax.div(cid, CPB) * (H * W)
        ws = []
        for g in range(NG):
            _, fx, fy = cell(p, base, g)
            gx = 1.0 - fx
            gy = 1.0 - fy
            ws.append((gx * gy, fx * gy, gx * fy, fx * fy))

        def n_body(n2, c2):
            lane = jnp.full((16,), 0, jnp.int32) + n2
            for g in range(NG):
                n = g * 16 + n2
                k, off = divmod(g * 16, SPLIT)
                gb = gbufs[p][k]
                r = n - k * SPLIT
                wa, wc, wb, wd = [
                    jnp.take_along_axis(w, lane, axis=0,
                                        mode="promise_in_bounds")
                    for w in ws[g]]
                def halves(v32):
                    w = plsc.bitcast(v32, jnp.int32)
                    e = plsc.bitcast(lax.shift_left(w, 16), jnp.float32)
                    o_ = plsc.bitcast(
                        jnp.bitwise_and(w, jnp.int32(-65536)), jnp.float32)
                    return e, o_

                for blk in range(C // 32):
                    o = blk * 32
                    ae, ao = halves(gb[r, pl.ds(o, 32)])
                    ce, co = halves(gb[r, pl.ds(C + o, 32)])
                    be, bo = halves(gb[r, pl.ds(2 * C + o, 32)])
                    de, do = halves(gb[r, pl.ds(3 * C + o, 32)])
                    obufs[p][n, pl.ds(o, 16)] = (
                        wa * ae + wc * ce + wb * be + wd * de)
                    obufs[p][n, pl.ds(o + 16, 16)] = (
                        wa * ao + wc * co + wb * bo + wd * do)
            return c2

        lax.fori_loop(0, 16, n_body, 0)
        pltpu.async_copy(obufs[p], out_slice, osems[p])

    fire(0, 0)

    def pair_body(t, carry):
        ci = t * NBUF
        first = t == 0

        @pl.when(ci + 1 < CPW)
        def _():
            fire(1, ci + 1)
        drain_blend(0, ci, first)

        @pl.when(ci + 2 < CPW)
        def _():
            fire(0, ci + 2)

        @pl.when(ci + 1 < CPW)
        def _():
            drain_blend(1, ci + 1, first)
        return carry

    lax.fori_loop(0, (CPW + NBUF - 1) // NBUF, pair_body, 0)
    for p in range(NBUF):
        ci = max(c for c in range(CPW) if c % NBUF == p)
        cid = wid * CPW + ci
        pltpu.make_async_copy(
            obufs[p], out_ref.at[pl.ds(cid * CHUNK, CHUNK)], osems[p]).wait()


@functools.partial(
    pl.kernel,
    out_type=jax.ShapeDtypeStruct((P, C), jnp.float32),
    mesh=_mesh,
    compiler_params=pltpu.CompilerParams(use_tc_tiling_on_sc=False,
                                         needs_layout_passes=False),
    scratch_types=[
        [[pltpu.VMEM((SPLIT,), jnp.int32)] * 2] * NBUF,
        [[pltpu.VMEM((SPLIT, D), jnp.bfloat16)] * 2] * NBUF,
        [pltpu.VMEM((CHUNK, C), jnp.float32)] * NBUF,
        [pltpu.VMEM((CHUNK,), jnp.float32)] * NBUF,
        [pltpu.VMEM((CHUNK,), jnp.float32)] * NBUF,
        [pltpu.SemaphoreType.DMA] * NBUF,
        [pltpu.SemaphoreType.DMA] * NBUF,
    ],
)
def _stn_sc(im_ref, xs_ref, ys_ref, out_ref,
            idx_vs, gbufs, obufs, x_vs, y_vs, gsems, osems):
    _stn_body(im_ref, xs_ref, ys_ref, out_ref,
              idx_vs, gbufs, obufs, x_vs, y_vs, gsems, osems)


def kernel(conv_input, theta_xy, theta_rt, theta_zm):
    im_flat = conv_input.reshape(P, C)
    im_sw = (im_flat.astype(jnp.bfloat16)
             .reshape(P, C // 32, 2, 16).transpose(0, 1, 3, 2).reshape(P, C))
    im_pad = jnp.pad(im_sw, ((0, W + 1), (0, 0)))
    tbl = jnp.concatenate(
        [im_pad[0:P], im_pad[1:P + 1], im_pad[W:P + W],
         im_pad[W + 1:P + W + 1]], axis=1)
    theta = theta_xy.reshape(-1, 2, 3)
    x_t, y_t = jnp.meshgrid(jnp.linspace(-1.0, 1.0, W),
                            jnp.linspace(-1.0, 1.0, H))
    grid = jnp.concatenate(
        [x_t.reshape(1, -1), y_t.reshape(1, -1),
         jnp.ones((1, H * W), dtype=jnp.float32)], axis=0)
    grid = jnp.broadcast_to(grid, (B, 3, H * W))
    T_g = jnp.matmul(theta, grid)
    xs = T_g[:, 0, :].reshape(-1)
    ys = T_g[:, 1, :].reshape(-1)
    out = _stn_sc(tbl, xs, ys)
    return out.reshape(B, H, W, C)


# speedup vs baseline: 1.1374x; 1.1374x over previous
import functools

import jax
import jax.numpy as jnp
from jax import lax
from jax.experimental import pallas as pl
from jax.experimental.pallas import tpu as pltpu
from jax.experimental.pallas import tpu_sc as plsc

B, H, W, C = 4, 224, 224, 96
P = B * H * W
D = 4 * C
CHUNK = 128
NW = 32
CHUNKS = P // CHUNK
CPB = (H * W) // CHUNK
CPW = CHUNKS // NW
NG = CHUNK // 16
NBUF = 2
SPLIT = 64

_mesh = plsc.VectorSubcoreMesh(core_axis_name="c", subcore_axis_name="s")


def _stn_body(im_ref, xs_ref, ys_ref, out_ref,
              idx_vs, gbufs, obufs, x_vs, y_vs, gsems, osems):
    wid = lax.axis_index("s") * 2 + lax.axis_index("c")

    def cell(p, base, g):
        sl = pl.ds(g * 16, 16)
        x = jnp.clip(x_vs[p][sl], -1.0, 1.0)
        y = jnp.clip(y_vs[p][sl], -1.0, 1.0)
        px = (x + 1.0) * ((W - 1) * 0.5)
        py = (y + 1.0) * ((H - 1) * 0.5)
        x0 = px.astype(jnp.int32)
        y0 = py.astype(jnp.int32)
        fx = px - x0.astype(jnp.float32)
        fy = py - y0.astype(jnp.float32)
        return base + y0 * W + x0, fx, fy

    def fire(p, ci):
        cid = wid * CPW + ci
        base = lax.div(cid, CPB) * (H * W)
        sl_in = pl.ds(cid * CHUNK, CHUNK)
        pltpu.sync_copy(xs_ref.at[sl_in], x_vs[p])
        pltpu.sync_copy(ys_ref.at[sl_in], y_vs[p])
        for g in range(NG):
            idx, _, _ = cell(p, base, g)
            k, off = divmod(g * 16, SPLIT)
            idx_vs[p][k][pl.ds(off, 16)] = idx
        for k in range(2):
            pltpu.async_copy(im_ref.at[idx_vs[p][k]], gbufs[p][k], gsems[p])

    def drain_blend(p, ci, first):
        for k in range(2):
            pltpu.make_async_copy(im_ref.at[idx_vs[p][k]], gbufs[p][k],
                                  gsems[p]).wait()
        cid = wid * CPW + ci
        out_slice = out_ref.at[pl.ds(cid * CHUNK, CHUNK)]

        @pl.when(jnp.logical_not(first))
        def _():
            pltpu.make_async_copy(obufs[p], out_slice, osems[p]).wait()
        base = lax.div(cid, CPB) * (H * W)
        ws = []
        for g in range(NG):
            _, fx, fy = cell(p, base, g)
            gx = 1.0 - fx
            gy = 1.0 - fy
            ws.append((gx * gy, fx * gy, gx * fy, fx * fy))

        def n_body(n2, c2):
            lane = jnp.full((16,), 0, jnp.int32) + n2
            for g in range(NG):
                n = g * 16 + n2
                k, off = divmod(g * 16, SPLIT)
                gb = gbufs[p][k]
                r = n - k * SPLIT
                wa, wc, wb, wd = [
                    jnp.take_along_axis(w, lane, axis=0,
                                        mode="promise_in_bounds")
                    for w in ws[g]]
                def halves(v32):
                    w = plsc.bitcast(v32, jnp.int32)
                    e = plsc.bitcast(lax.shift_left(w, 16), jnp.float32)
                    o_ = plsc.bitcast(
                        jnp.bitwise_and(w, jnp.int32(-65536)), jnp.float32)
                    return e, o_

                row = jnp.full((16,), g * 16, jnp.int32) + n2
                for blk in range(C // 32):
                    o = blk * 32
                    ae, ao = halves(gb[r, pl.ds(o, 32)])
                    ce, co = halves(gb[r, pl.ds(C + o, 32)])
                    be, bo = halves(gb[r, pl.ds(2 * C + o, 32)])
                    de, do = halves(gb[r, pl.ds(3 * C + o, 32)])
                    col = o + 2 * lax.iota(jnp.int32, 16)
                    plsc.store_scatter(
                        obufs[p], [row, col],
                        wa * ae + wc * ce + wb * be + wd * de)
                    plsc.store_scatter(
                        obufs[p], [row, col + 1],
                        wa * ao + wc * co + wb * bo + wd * do)
            return c2

        lax.fori_loop(0, 16, n_body, 0)
        pltpu.async_copy(obufs[p], out_slice, osems[p])

    fire(0, 0)

    def pair_body(t, carry):
        ci = t * NBUF
        first = t == 0

        @pl.when(ci + 1 < CPW)
        def _():
            fire(1, ci + 1)
        drain_blend(0, ci, first)

        @pl.when(ci + 2 < CPW)
        def _():
            fire(0, ci + 2)

        @pl.when(ci + 1 < CPW)
        def _():
            drain_blend(1, ci + 1, first)
        return carry

    lax.fori_loop(0, (CPW + NBUF - 1) // NBUF, pair_body, 0)
    for p in range(NBUF):
        ci = max(c for c in range(CPW) if c % NBUF == p)
        cid = wid * CPW + ci
        pltpu.make_async_copy(
            obufs[p], out_ref.at[pl.ds(cid * CHUNK, CHUNK)], osems[p]).wait()


@functools.partial(
    pl.kernel,
    out_type=jax.ShapeDtypeStruct((P, C), jnp.float32),
    mesh=_mesh,
    compiler_params=pltpu.CompilerParams(use_tc_tiling_on_sc=False,
                                         needs_layout_passes=False),
    scratch_types=[
        [[pltpu.VMEM((SPLIT,), jnp.int32)] * 2] * NBUF,
        [[pltpu.VMEM((SPLIT, D), jnp.bfloat16)] * 2] * NBUF,
        [pltpu.VMEM((CHUNK, C), jnp.float32)] * NBUF,
        [pltpu.VMEM((CHUNK,), jnp.float32)] * NBUF,
        [pltpu.VMEM((CHUNK,), jnp.float32)] * NBUF,
        [pltpu.SemaphoreType.DMA] * NBUF,
        [pltpu.SemaphoreType.DMA] * NBUF,
    ],
)
def _stn_sc(im_ref, xs_ref, ys_ref, out_ref,
            idx_vs, gbufs, obufs, x_vs, y_vs, gsems, osems):
    _stn_body(im_ref, xs_ref, ys_ref, out_ref,
              idx_vs, gbufs, obufs, x_vs, y_vs, gsems, osems)


def kernel(conv_input, theta_xy, theta_rt, theta_zm):
    im_flat = conv_input.reshape(P, C)
    im_pad = jnp.pad(im_flat.astype(jnp.bfloat16), ((0, W + 1), (0, 0)))
    tbl = jnp.concatenate(
        [im_pad[0:P], im_pad[1:P + 1], im_pad[W:P + W],
         im_pad[W + 1:P + W + 1]], axis=1)
    theta = theta_xy.reshape(-1, 2, 3)
    x_t, y_t = jnp.meshgrid(jnp.linspace(-1.0, 1.0, W),
                            jnp.linspace(-1.0, 1.0, H))
    grid = jnp.concatenate(
        [x_t.reshape(1, -1), y_t.reshape(1, -1),
         jnp.ones((1, H * W), dtype=jnp.float32)], axis=0)
    grid = jnp.broadcast_to(grid, (B, 3, H * W))
    T_g = jnp.matmul(theta, grid)
    xs = T_g[:, 0, :].reshape(-1)
    ys = T_g[:, 1, :].reshape(-1)
    out = _stn_sc(tbl, xs, ys)
    return out.reshape(B, H, W, C)


# speedup vs baseline: 1.1969x; 1.0524x over previous
import functools

import jax
import jax.numpy as jnp
from jax import lax
from jax.experimental import pallas as pl
from jax.experimental.pallas import tpu as pltpu
from jax.experimental.pallas import tpu_sc as plsc

B, H, W, C = 4, 224, 224, 96
P = B * H * W
D = 4 * C
CHUNK = 112
CPR = W // CHUNK
NW = 32
CHUNKS = P // CHUNK
CPW = CHUNKS // NW
NG = CHUNK // 16
NBUF = 2
SPLIT = 64

_mesh = plsc.VectorSubcoreMesh(core_axis_name="c", subcore_axis_name="s")


def _stn_body(im_ref, xs_ref, ys_ref, out_ref,
              idx_vs, gbufs, obufs, x_vs, y_vs, gsems, osems):
    wid = lax.axis_index("s") * 2 + lax.axis_index("c")

    def cell(p, base, g):
        sl = pl.ds(g * 16, 16)
        x = jnp.clip(x_vs[p][sl], -1.0, 1.0)
        y = jnp.clip(y_vs[p][sl], -1.0, 1.0)
        px = (x + 1.0) * ((W - 1) * 0.5)
        py = (y + 1.0) * ((H - 1) * 0.5)
        x0 = px.astype(jnp.int32)
        y0 = py.astype(jnp.int32)
        fx = px - x0.astype(jnp.float32)
        fy = py - y0.astype(jnp.float32)
        return base + y0 * W + x0, fx, fy

    def fire(p, ci):
        cid = wid * CPW + ci
        b = lax.div(cid, H * CPR)
        base = b * (H * W)
        sl_in = pl.ds(cid * CHUNK, CHUNK)
        pltpu.sync_copy(xs_ref.at[sl_in], x_vs[p])
        pltpu.sync_copy(ys_ref.at[sl_in], y_vs[p])
        for g in range(NG):
            idx, _, _ = cell(p, base, g)
            if g < SPLIT // 16:
                idx_vs[p][0][pl.ds(g * 16, 16)] = idx
            else:
                idx_vs[p][1][pl.ds(g * 16 - SPLIT, 16)] = idx
        for k in range(2):
            pltpu.async_copy(im_ref.at[idx_vs[p][k]], gbufs[p][k], gsems[p])

    def drain_blend(p, ci, first):
        for k in range(2):
            pltpu.make_async_copy(im_ref.at[idx_vs[p][k]], gbufs[p][k],
                                  gsems[p]).wait()
        cid = wid * CPW + ci
        out_slice = out_ref.at[pl.ds(cid * CHUNK, CHUNK)]

        @pl.when(jnp.logical_not(first))
        def _():
            pltpu.make_async_copy(obufs[p], out_slice, osems[p]).wait()
        b = lax.div(cid, H * CPR)
        base = b * (H * W)
        ws = []
        for g in range(NG):
            _, fx, fy = cell(p, base, g)
            gx = 1.0 - fx
            gy = 1.0 - fy
            ws.append((gx * gy, fx * gy, gx * fy, fx * fy))

        def n_body(n2, c2):
            lane = jnp.full((16,), 0, jnp.int32) + n2
            for g in range(NG):
                n = g * 16 + n2
                gb = gbufs[p][0] if g < SPLIT // 16 else gbufs[p][1]
                r = n if g < SPLIT // 16 else n - SPLIT
                wa, wc, wb, wd = [
                    jnp.take_along_axis(w, lane, axis=0,
                                        mode="promise_in_bounds")
                    for w in ws[g]]
                for cb in range(C // 16):
                    o = cb * 16
                    obufs[p][n, pl.ds(o, 16)] = (
                        wa * gb[r, pl.ds(o, 16)]
                        + wc * gb[r, pl.ds(C + o, 16)]
                        + wb * gb[r, pl.ds(2 * C + o, 16)]
                        + wd * gb[r, pl.ds(3 * C + o, 16)])
            return c2

        lax.fori_loop(0, 16, n_body, 0)
        pltpu.async_copy(obufs[p], out_slice, osems[p])

    fire(0, 0)

    def pair_body(t, carry):
        ci = t * NBUF
        first = t == 0

        @pl.when(ci + 1 < CPW)
        def _():
            fire(1, ci + 1)
        drain_blend(0, ci, first)

        @pl.when(ci + 2 < CPW)
        def _():
            fire(0, ci + 2)

        @pl.when(ci + 1 < CPW)
        def _():
            drain_blend(1, ci + 1, first)
        return carry

    lax.fori_loop(0, (CPW + NBUF - 1) // NBUF, pair_body, 0)
    for p in range(NBUF):
        ci = CPW - NBUF + p
        cid = wid * CPW + ci
        pltpu.make_async_copy(
            obufs[p], out_ref.at[pl.ds(cid * CHUNK, CHUNK)], osems[p]).wait()


@functools.partial(
    pl.kernel,
    out_type=jax.ShapeDtypeStruct((P, C), jnp.float32),
    mesh=_mesh,
    compiler_params=pltpu.CompilerParams(use_tc_tiling_on_sc=True),
    scratch_types=[
        [[pltpu.VMEM((SPLIT,), jnp.int32),
          pltpu.VMEM((CHUNK - SPLIT,), jnp.int32)]] * NBUF,
        [[pltpu.VMEM((SPLIT, D), jnp.float32),
          pltpu.VMEM((CHUNK - SPLIT, D), jnp.float32)]] * NBUF,
        [pltpu.VMEM((CHUNK, C), jnp.float32)] * NBUF,
        [pltpu.VMEM((CHUNK,), jnp.float32)] * NBUF,
        [pltpu.VMEM((CHUNK,), jnp.float32)] * NBUF,
        [pltpu.SemaphoreType.DMA] * NBUF,
        [pltpu.SemaphoreType.DMA] * NBUF,
    ],
)
def _stn_sc(im_ref, xs_ref, ys_ref, out_ref,
            idx_vs, gbufs, obufs, x_vs, y_vs, gsems, osems):
    _stn_body(im_ref, xs_ref, ys_ref, out_ref,
              idx_vs, gbufs, obufs, x_vs, y_vs, gsems, osems)


def kernel(conv_input, theta_xy, theta_rt, theta_zm):
    im_flat = conv_input.reshape(P, C)
    im_pad = jnp.pad(im_flat, ((0, W + 1), (0, 0)))
    tbl = jnp.concatenate(
        [im_pad[0:P], im_pad[1:P + 1], im_pad[W:P + W],
         im_pad[W + 1:P + W + 1]], axis=1)
    theta = theta_xy.reshape(-1, 2, 3)
    x_t, y_t = jnp.meshgrid(jnp.linspace(-1.0, 1.0, W),
                            jnp.linspace(-1.0, 1.0, H))
    grid = jnp.concatenate(
        [x_t.reshape(1, -1), y_t.reshape(1, -1),
         jnp.ones((1, H * W), dtype=jnp.float32)], axis=0)
    grid = jnp.broadcast_to(grid, (B, 3, H * W))
    T_g = jnp.matmul(theta, grid)
    xs = T_g[:, 0, :].reshape(-1)
    ys = T_g[:, 1, :].reshape(-1)
    out = _stn_sc(tbl, xs, ys)
    return out.reshape(B, H, W, C)
